# baseline (speedup 1.0000x reference)
import functools

import jax
import jax.numpy as jnp
from jax import lax
from jax.experimental import pallas as pl
from jax.experimental.pallas import tpu as pltpu
from jax.experimental.pallas import tpu_sc as plsc

B, S = 1024, 50
N = B * S
HIDDEN = 1024
EPS = 1e-05

NC, NS = 2, 16
NW = NC * NS
TOK_PER_W = N // NW
CHUNK = 80
NCHUNK = TOK_PER_W // CHUNK
BIG_WIDTHS = (256, 128)

TOK_TILE = 1024
GRID = N // TOK_TILE


def _sc_gather(sp_ids_a, mv_ids_a, sp_t, mv_t):
    mesh = plsc.VectorSubcoreMesh(core_axis_name="c", subcore_axis_name="s")

    @functools.partial(
        pl.kernel,
        mesh=mesh,
        out_type=[jax.ShapeDtypeStruct((N, w), jnp.float32) for w in BIG_WIDTHS],
        scratch_types=[
            pltpu.VMEM((TOK_PER_W,), jnp.int32),
            pltpu.VMEM((TOK_PER_W,), jnp.int32),
            pltpu.VMEM((CHUNK, 256), jnp.float32),
            pltpu.VMEM((CHUNK, 256), jnp.float32),
            pltpu.VMEM((CHUNK, 128), jnp.float32),
            pltpu.VMEM((CHUNK, 128), jnp.float32),
            pltpu.SemaphoreType.DMA,
            pltpu.SemaphoreType.DMA,
        ],
    )
    def k(sp_ids, mv_ids, sp_hbm, mv_hbm, o_sp, o_mv,
          i_sp, i_mv, b_sp0, b_sp1, b_mv0, b_mv1, gsem, ssem):
        wid = lax.axis_index("s") * NC + lax.axis_index("c")
        base_w = pl.multiple_of(wid * TOK_PER_W, TOK_PER_W)
        pltpu.sync_copy(sp_ids.at[wid], i_sp)
        pltpu.sync_copy(mv_ids.at[wid], i_mv)
        idxs = (i_sp, i_mv)
        tables = (sp_hbm, mv_hbm)
        bufs = ((b_sp0, b_sp1), (b_mv0, b_mv1))
        outs = (o_sp, o_mv)

        def gather(kk, slot):
            off = pl.multiple_of(kk * CHUNK, CHUNK)
            return [pltpu.async_copy(
                tables[t].at[idxs[t].at[pl.ds(off, CHUNK)]],
                bufs[t][slot], gsem) for t in range(2)]

        def store(kk, slot):
            off = pl.multiple_of(kk * CHUNK, CHUNK)
            return [pltpu.async_copy(
                bufs[t][slot], outs[t].at[pl.ds(base_w + off, CHUNK)], ssem)
                for t in range(2)]

        gath_next = gather(0, 0)
        stores = [[], []]
        for kk in range(NCHUNK):
            slot = kk % 2
            nslot = (kk + 1) % 2
            cur_g = gath_next
            if kk + 1 < NCHUNK:
                for x in stores[nslot]:
                    x.wait()
                gath_next = gather(kk + 1, nslot)
            for x in cur_g:
                x.wait()
            stores[slot] = store(kk, slot)
        for sl in (0, 1):
            for x in stores[sl]:
                x.wait()

    return k(sp_ids_a, mv_ids_a, sp_t, mv_t)


def _tc_body(st_ids_ref, we_ids_ref, te_ids_ref, po_ids_ref,
             it_ids_ref, ab_ids_ref,
             sp_ref, mv_ref, hp_ref, bo_ref,
             st_t_ref, we_t_ref, te_t_ref, po_t_ref, it_t_ref, ab_t_ref,
             hp_W_ref, hp_b_ref, boost_W_ref, boost_b_ref,
             wproj_ref, proj_b_ref, gamma_ref, beta_ref, out_ref):
    f32 = jnp.float32
    bf16 = jnp.bfloat16

    def onehot_emb(ids_ref, tbl_ref, vocab):
        ids = ids_ref[...]
        oh = (ids == lax.broadcasted_iota(
            jnp.int32, (TOK_TILE, vocab), 1)).astype(bf16)
        return jnp.dot(oh, tbl_ref[...].astype(bf16),
                       preferred_element_type=f32)

    st_emb = onehot_emb(st_ids_ref, st_t_ref, 8)
    we_emb = onehot_emb(we_ids_ref, we_t_ref, 10)
    te_emb = onehot_emb(te_ids_ref, te_t_ref, 5)
    po_emb = onehot_emb(po_ids_ref, po_t_ref, 12)
    it_emb = onehot_emb(it_ids_ref, it_t_ref, 400)
    ab_emb = onehot_emb(ab_ids_ref, ab_t_ref, 300)
    hp_emb = hp_ref[...] * hp_W_ref[...] + hp_b_ref[...][None, :]
    bo_emb = jnp.dot(bo_ref[...], boost_W_ref[...],
                     preferred_element_type=f32) + boost_b_ref[...][None, :]

    combined = jnp.concatenate([
        sp_ref[...], mv_ref[...], it_emb, ab_emb,
        hp_emb, bo_emb, st_emb, we_emb, te_emb, po_emb], axis=1).astype(bf16)

    acc = jnp.dot(combined, wproj_ref[...], preferred_element_type=f32)
    acc = acc + proj_b_ref[...][None, :]
    mean = jnp.mean(acc, axis=1, keepdims=True)
    cen = acc - mean
    var = jnp.mean(cen * cen, axis=1, keepdims=True)
    y = cen * lax.rsqrt(var + EPS)
    out_ref[...] = y * gamma_ref[...][None, :] + beta_ref[...][None, :]


def _full(shape):
    nd = len(shape)
    return pl.BlockSpec(shape, lambda i: (0,) * nd)


def kernel(species_ids, move_ids, item_ids, ability_ids, hp_values, stat_boosts,
           status_ids, weather_ids, terrain_ids, position_ids,
           species_table, move_table, item_table, ability_table,
           hp_W, hp_b, boost_W, boost_b,
           status_table, weather_table, terrain_table, position_table,
           proj_W, proj_b, ln_gamma, ln_beta):
    i32 = jnp.int32
    sp_idw = species_ids.T.reshape(NW, TOK_PER_W).astype(i32)
    mv_idw = move_ids.T.reshape(NW, TOK_PER_W).astype(i32)

    sp_e, mv_e = _sc_gather(sp_idw, mv_idw, species_table, move_table)

    st2 = status_ids.T.reshape(N, 1).astype(i32)
    we2 = weather_ids.T.reshape(N, 1).astype(i32)
    te2 = terrain_ids.T.reshape(N, 1).astype(i32)
    po2 = position_ids.T.reshape(N, 1).astype(i32)
    it2 = item_ids.T.reshape(N, 1).astype(i32)
    ab2 = ability_ids.T.reshape(N, 1).astype(i32)
    hp2 = hp_values.T.reshape(N, 1)
    bo2 = stat_boosts.transpose(1, 0, 2).reshape(N, 7)
    wproj_bf = proj_W.astype(jnp.bfloat16)

    ids_spec = pl.BlockSpec((TOK_TILE, 1), lambda i: (i, 0))

    out = pl.pallas_call(
        _tc_body,
        grid=(GRID,),
        in_specs=[
            ids_spec, ids_spec, ids_spec, ids_spec, ids_spec, ids_spec,
            pl.BlockSpec((TOK_TILE, 256), lambda i: (i, 0)),
            pl.BlockSpec((TOK_TILE, 128), lambda i: (i, 0)),
            ids_spec,
            pl.BlockSpec((TOK_TILE, 7), lambda i: (i, 0)),
            _full((8, 32)), _full((10, 32)), _full((5, 32)), _full((12, 64)),
            _full((400, 64)), _full((300, 64)),
            _full((1, 32)), _full((32,)), _full((7, 32)), _full((32,)),
            _full((736, HIDDEN)), _full((HIDDEN,)),
            _full((HIDDEN,)), _full((HIDDEN,)),
        ],
        out_specs=pl.BlockSpec((TOK_TILE, HIDDEN), lambda i: (i, 0)),
        out_shape=jax.ShapeDtypeStruct((N, HIDDEN), jnp.float32),
        compiler_params=pltpu.CompilerParams(
            dimension_semantics=("arbitrary",)),
    )(st2, we2, te2, po2, it2, ab2, sp_e, mv_e, hp2, bo2,
      status_table, weather_table, terrain_table, position_table,
      item_table, ability_table,
      hp_W, hp_b, boost_W, boost_b,
      wproj_bf, proj_b, ln_gamma, ln_beta)

    return out.reshape(S, B, HIDDEN).transpose(1, 0, 2)


# speedup vs baseline: 1.2575x; 1.2575x over previous
import functools

import jax
import jax.numpy as jnp
from jax import lax
from jax.experimental import pallas as pl
from jax.experimental.pallas import tpu as pltpu
from jax.experimental.pallas import tpu_sc as plsc

B, S = 1024, 50
N = B * S
HIDDEN = 1024
EPS = 1e-05

NC, NS = 2, 16
NW = NC * NS
TOK_PER_W = N // NW
CHUNK = 80
NCHUNK = TOK_PER_W // CHUNK
BIG_WIDTHS = (256, 128)

TOK_TILE = 1024
GRID = N // TOK_TILE


def _sc_gather(sp_ids_a, mv_ids_a, sp_t, mv_t):
    mesh = plsc.VectorSubcoreMesh(core_axis_name="c", subcore_axis_name="s")

    @functools.partial(
        pl.kernel,
        mesh=mesh,
        out_type=[jax.ShapeDtypeStruct((N, w), jnp.float32) for w in BIG_WIDTHS],
        scratch_types=[
            pltpu.VMEM((TOK_PER_W,), jnp.int32),
            pltpu.VMEM((TOK_PER_W,), jnp.int32),
            pltpu.VMEM((CHUNK, 256), jnp.float32),
            pltpu.VMEM((CHUNK, 256), jnp.float32),
            pltpu.VMEM((CHUNK, 128), jnp.float32),
            pltpu.VMEM((CHUNK, 128), jnp.float32),
            pltpu.SemaphoreType.DMA,
            pltpu.SemaphoreType.DMA,
        ],
    )
    def k(sp_ids, mv_ids, sp_hbm, mv_hbm, o_sp, o_mv,
          i_sp, i_mv, b_sp0, b_sp1, b_mv0, b_mv1, gsem, ssem):
        wid = lax.axis_index("s") * NC + lax.axis_index("c")
        base_w = pl.multiple_of(wid * TOK_PER_W, TOK_PER_W)
        pltpu.sync_copy(sp_ids.at[wid], i_sp)
        pltpu.sync_copy(mv_ids.at[wid], i_mv)
        idxs = (i_sp, i_mv)
        tables = (sp_hbm, mv_hbm)
        bufs = ((b_sp0, b_sp1), (b_mv0, b_mv1))
        outs = (o_sp, o_mv)

        def gather(kk, slot):
            off = pl.multiple_of(kk * CHUNK, CHUNK)
            return [pltpu.async_copy(
                tables[t].at[idxs[t].at[pl.ds(off, CHUNK)]],
                bufs[t][slot], gsem) for t in range(2)]

        def store(kk, slot):
            off = pl.multiple_of(kk * CHUNK, CHUNK)
            return [pltpu.async_copy(
                bufs[t][slot], outs[t].at[pl.ds(base_w + off, CHUNK)], ssem)
                for t in range(2)]

        gath_next = gather(0, 0)
        stores = [[], []]
        for kk in range(NCHUNK):
            slot = kk % 2
            nslot = (kk + 1) % 2
            cur_g = gath_next
            if kk + 1 < NCHUNK:
                for x in stores[nslot]:
                    x.wait()
                gath_next = gather(kk + 1, nslot)
            for x in cur_g:
                x.wait()
            stores[slot] = store(kk, slot)
        for sl in (0, 1):
            for x in stores[sl]:
                x.wait()

    return k(sp_ids_a, mv_ids_a, sp_t, mv_t)


def _tc_body(st_ids_ref, we_ids_ref, te_ids_ref, po_ids_ref,
             it_ids_ref, ab_ids_ref,
             sp_ref, mv_ref, hp_ref, bo_ref,
             st_t_ref, we_t_ref, te_t_ref, po_t_ref, it_t_ref, ab_t_ref,
             hp_W_ref, hp_b_ref, boost_W_ref, boost_b_ref,
             wproj_ref, proj_b_ref, gamma_ref, beta_ref, out_ref):
    f32 = jnp.float32
    bf16 = jnp.bfloat16

    def onehot_emb(ids_ref, tbl_ref, vocab):
        ids = ids_ref[0, 0, :]
        oh = (ids[:, None] == lax.broadcasted_iota(
            jnp.int32, (TOK_TILE, vocab), 1)).astype(bf16)
        return jnp.dot(oh, tbl_ref[...].astype(bf16),
                       preferred_element_type=f32)

    st_emb = onehot_emb(st_ids_ref, st_t_ref, 8)
    we_emb = onehot_emb(we_ids_ref, we_t_ref, 10)
    te_emb = onehot_emb(te_ids_ref, te_t_ref, 5)
    po_emb = onehot_emb(po_ids_ref, po_t_ref, 12)
    it_emb = onehot_emb(it_ids_ref, it_t_ref, 400)
    ab_emb = onehot_emb(ab_ids_ref, ab_t_ref, 300)
    hp_emb = hp_ref[...] * hp_W_ref[...] + hp_b_ref[...][None, :]
    bo_emb = jnp.dot(bo_ref[...], boost_W_ref[...],
                     preferred_element_type=f32) + boost_b_ref[...][None, :]

    combined = jnp.concatenate([
        sp_ref[...], mv_ref[...], it_emb, ab_emb,
        hp_emb, bo_emb, st_emb, we_emb, te_emb, po_emb], axis=1).astype(bf16)

    acc = jnp.dot(combined, wproj_ref[...], preferred_element_type=f32)
    acc = acc + proj_b_ref[...][None, :]
    mean = jnp.mean(acc, axis=1, keepdims=True)
    cen = acc - mean
    var = jnp.mean(cen * cen, axis=1, keepdims=True)
    y = cen * lax.rsqrt(var + EPS)
    out_ref[...] = y * gamma_ref[...][None, :] + beta_ref[...][None, :]


def _full(shape):
    nd = len(shape)
    return pl.BlockSpec(shape, lambda i: (0,) * nd)


def kernel(species_ids, move_ids, item_ids, ability_ids, hp_values, stat_boosts,
           status_ids, weather_ids, terrain_ids, position_ids,
           species_table, move_table, item_table, ability_table,
           hp_W, hp_b, boost_W, boost_b,
           status_table, weather_table, terrain_table, position_table,
           proj_W, proj_b, ln_gamma, ln_beta):
    i32 = jnp.int32
    sp_idw = species_ids.T.reshape(NW, TOK_PER_W).astype(i32)
    mv_idw = move_ids.T.reshape(NW, TOK_PER_W).astype(i32)

    sp_e, mv_e = _sc_gather(sp_idw, mv_idw, species_table, move_table)

    st2 = status_ids.T.reshape(GRID, 1, TOK_TILE).astype(i32)
    we2 = weather_ids.T.reshape(GRID, 1, TOK_TILE).astype(i32)
    te2 = terrain_ids.T.reshape(GRID, 1, TOK_TILE).astype(i32)
    po2 = position_ids.T.reshape(GRID, 1, TOK_TILE).astype(i32)
    it2 = item_ids.T.reshape(GRID, 1, TOK_TILE).astype(i32)
    ab2 = ability_ids.T.reshape(GRID, 1, TOK_TILE).astype(i32)
    hp2 = hp_values.T.reshape(N, 1)
    bo2 = stat_boosts.transpose(1, 0, 2).reshape(N, 7)
    wproj_bf = proj_W.astype(jnp.bfloat16)

    ids_spec = pl.BlockSpec((1, 1, TOK_TILE), lambda i: (i, 0, 0))

    out = pl.pallas_call(
        _tc_body,
        grid=(GRID,),
        in_specs=[
            ids_spec, ids_spec, ids_spec, ids_spec, ids_spec, ids_spec,
            pl.BlockSpec((TOK_TILE, 256), lambda i: (i, 0)),
            pl.BlockSpec((TOK_TILE, 128), lambda i: (i, 0)),
            pl.BlockSpec((TOK_TILE, 1), lambda i: (i, 0)),
            pl.BlockSpec((TOK_TILE, 7), lambda i: (i, 0)),
            _full((8, 32)), _full((10, 32)), _full((5, 32)), _full((12, 64)),
            _full((400, 64)), _full((300, 64)),
            _full((1, 32)), _full((32,)), _full((7, 32)), _full((32,)),
            _full((736, HIDDEN)), _full((HIDDEN,)),
            _full((HIDDEN,)), _full((HIDDEN,)),
        ],
        out_specs=pl.BlockSpec((TOK_TILE, HIDDEN), lambda i: (i, 0)),
        out_shape=jax.ShapeDtypeStruct((N, HIDDEN), jnp.float32),
        compiler_params=pltpu.CompilerParams(
            dimension_semantics=("arbitrary",)),
    )(st2, we2, te2, po2, it2, ab2, sp_e, mv_e, hp2, bo2,
      status_table, weather_table, terrain_table, position_table,
      item_table, ability_table,
      hp_W, hp_b, boost_W, boost_b,
      wproj_bf, proj_b, ln_gamma, ln_beta)

    return out.reshape(S, B, HIDDEN).transpose(1, 0, 2)


# speedup vs baseline: 1.2584x; 1.0007x over previous
import functools

import jax
import jax.numpy as jnp
from jax import lax
from jax.experimental import pallas as pl
from jax.experimental.pallas import tpu as pltpu
from jax.experimental.pallas import tpu_sc as plsc

B, S = 1024, 50
N = B * S
HIDDEN = 1024
EPS = 1e-05

NC, NS = 2, 16
NW = NC * NS
TOK_PER_W = N // NW
CHUNK = 80
NCHUNK = TOK_PER_W // CHUNK
BIG_WIDTHS = (256, 128)

TOK_TILE = 1024
GRID = N // TOK_TILE


def _sc_gather(sp_ids_a, mv_ids_a, sp_t, mv_t):
    mesh = plsc.VectorSubcoreMesh(core_axis_name="c", subcore_axis_name="s")

    @functools.partial(
        pl.kernel,
        mesh=mesh,
        out_type=[jax.ShapeDtypeStruct((N, w), jnp.float32) for w in BIG_WIDTHS],
        scratch_types=[
            pltpu.VMEM((TOK_PER_W,), jnp.int32),
            pltpu.VMEM((TOK_PER_W,), jnp.int32),
            pltpu.VMEM((CHUNK, 256), jnp.float32),
            pltpu.VMEM((CHUNK, 256), jnp.float32),
            pltpu.VMEM((CHUNK, 256), jnp.float32),
            pltpu.VMEM((CHUNK, 128), jnp.float32),
            pltpu.VMEM((CHUNK, 128), jnp.float32),
            pltpu.VMEM((CHUNK, 128), jnp.float32),
            pltpu.SemaphoreType.DMA,
            pltpu.SemaphoreType.DMA,
        ],
    )
    def k(sp_ids, mv_ids, sp_hbm, mv_hbm, o_sp, o_mv,
          i_sp, i_mv, b_sp0, b_sp1, b_sp2, b_mv0, b_mv1, b_mv2, gsem, ssem):
        wid = lax.axis_index("s") * NC + lax.axis_index("c")
        base_w = pl.multiple_of(wid * TOK_PER_W, TOK_PER_W)
        pltpu.sync_copy(sp_ids.at[wid], i_sp)
        pltpu.sync_copy(mv_ids.at[wid], i_mv)
        idxs = (i_sp, i_mv)
        tables = (sp_hbm, mv_hbm)
        bufs = ((b_sp0, b_sp1, b_sp2), (b_mv0, b_mv1, b_mv2))
        outs = (o_sp, o_mv)
        DEPTH = 3

        def gather(kk, slot):
            off = pl.multiple_of(kk * CHUNK, CHUNK)
            return [pltpu.async_copy(
                tables[t].at[idxs[t].at[pl.ds(off, CHUNK)]],
                bufs[t][slot], gsem) for t in range(2)]

        def store(kk, slot):
            off = pl.multiple_of(kk * CHUNK, CHUNK)
            return [pltpu.async_copy(
                bufs[t][slot], outs[t].at[pl.ds(base_w + off, CHUNK)], ssem)
                for t in range(2)]

        gaths = [None] * NCHUNK
        stores = [[] for _ in range(DEPTH)]
        for j in range(min(DEPTH - 1, NCHUNK)):
            gaths[j] = gather(j, j % DEPTH)
        for kk in range(NCHUNK):
            slot = kk % DEPTH
            nxt = kk + DEPTH - 1
            if nxt < NCHUNK:
                for x in stores[nxt % DEPTH]:
                    x.wait()
                stores[nxt % DEPTH] = []
                gaths[nxt] = gather(nxt, nxt % DEPTH)
            for x in gaths[kk]:
                x.wait()
            stores[slot] = store(kk, slot)
        for sl in range(DEPTH):
            for x in stores[sl]:
                x.wait()

    return k(sp_ids_a, mv_ids_a, sp_t, mv_t)


def _tc_body(st_ids_ref, we_ids_ref, te_ids_ref, po_ids_ref,
             it_ids_ref, ab_ids_ref,
             sp_ref, mv_ref, hp_ref, bo_ref,
             st_t_ref, we_t_ref, te_t_ref, po_t_ref, it_t_ref, ab_t_ref,
             hp_W_ref, hp_b_ref, boost_W_ref, boost_b_ref,
             wproj_ref, proj_b_ref, gamma_ref, beta_ref, out_ref):
    f32 = jnp.float32
    bf16 = jnp.bfloat16

    def onehot_emb(ids_ref, tbl_ref, vocab):
        ids = ids_ref[0, 0, :]
        oh = (ids[:, None] == lax.broadcasted_iota(
            jnp.int32, (TOK_TILE, vocab), 1)).astype(bf16)
        return jnp.dot(oh, tbl_ref[...].astype(bf16),
                       preferred_element_type=f32)

    st_emb = onehot_emb(st_ids_ref, st_t_ref, 8)
    we_emb = onehot_emb(we_ids_ref, we_t_ref, 10)
    te_emb = onehot_emb(te_ids_ref, te_t_ref, 5)
    po_emb = onehot_emb(po_ids_ref, po_t_ref, 12)
    it_emb = onehot_emb(it_ids_ref, it_t_ref, 400)
    ab_emb = onehot_emb(ab_ids_ref, ab_t_ref, 300)
    hp_emb = hp_ref[...] * hp_W_ref[...] + hp_b_ref[...][None, :]
    bo_emb = jnp.dot(bo_ref[...], boost_W_ref[...],
                     preferred_element_type=f32) + boost_b_ref[...][None, :]

    combined = jnp.concatenate([
        sp_ref[...], mv_ref[...], it_emb, ab_emb,
        hp_emb, bo_emb, st_emb, we_emb, te_emb, po_emb], axis=1).astype(bf16)

    acc = jnp.dot(combined, wproj_ref[...], preferred_element_type=f32)
    acc = acc + proj_b_ref[...][None, :]
    mean = jnp.mean(acc, axis=1, keepdims=True)
    cen = acc - mean
    var = jnp.mean(cen * cen, axis=1, keepdims=True)
    y = cen * lax.rsqrt(var + EPS)
    out_ref[...] = y * gamma_ref[...][None, :] + beta_ref[...][None, :]


def _full(shape):
    nd = len(shape)
    return pl.BlockSpec(shape, lambda i: (0,) * nd)


def kernel(species_ids, move_ids, item_ids, ability_ids, hp_values, stat_boosts,
           status_ids, weather_ids, terrain_ids, position_ids,
           species_table, move_table, item_table, ability_table,
           hp_W, hp_b, boost_W, boost_b,
           status_table, weather_table, terrain_table, position_table,
           proj_W, proj_b, ln_gamma, ln_beta):
    i32 = jnp.int32
    sp_idw = species_ids.T.reshape(NW, TOK_PER_W).astype(i32)
    mv_idw = move_ids.T.reshape(NW, TOK_PER_W).astype(i32)

    sp_e, mv_e = _sc_gather(sp_idw, mv_idw, species_table, move_table)

    st2 = status_ids.T.reshape(GRID, 1, TOK_TILE).astype(i32)
    we2 = weather_ids.T.reshape(GRID, 1, TOK_TILE).astype(i32)
    te2 = terrain_ids.T.reshape(GRID, 1, TOK_TILE).astype(i32)
    po2 = position_ids.T.reshape(GRID, 1, TOK_TILE).astype(i32)
    it2 = item_ids.T.reshape(GRID, 1, TOK_TILE).astype(i32)
    ab2 = ability_ids.T.reshape(GRID, 1, TOK_TILE).astype(i32)
    hp2 = hp_values.T.reshape(N, 1)
    bo2 = stat_boosts.transpose(1, 0, 2).reshape(N, 7)
    wproj_bf = proj_W.astype(jnp.bfloat16)

    ids_spec = pl.BlockSpec((1, 1, TOK_TILE), lambda i: (i, 0, 0))

    out = pl.pallas_call(
        _tc_body,
        grid=(GRID,),
        in_specs=[
            ids_spec, ids_spec, ids_spec, ids_spec, ids_spec, ids_spec,
            pl.BlockSpec((TOK_TILE, 256), lambda i: (i, 0)),
            pl.BlockSpec((TOK_TILE, 128), lambda i: (i, 0)),
            pl.BlockSpec((TOK_TILE, 1), lambda i: (i, 0)),
            pl.BlockSpec((TOK_TILE, 7), lambda i: (i, 0)),
            _full((8, 32)), _full((10, 32)), _full((5, 32)), _full((12, 64)),
            _full((400, 64)), _full((300, 64)),
            _full((1, 32)), _full((32,)), _full((7, 32)), _full((32,)),
            _full((736, HIDDEN)), _full((HIDDEN,)),
            _full((HIDDEN,)), _full((HIDDEN,)),
        ],
        out_specs=pl.BlockSpec((TOK_TILE, HIDDEN), lambda i: (i, 0)),
        out_shape=jax.ShapeDtypeStruct((N, HIDDEN), jnp.float32),
        compiler_params=pltpu.CompilerParams(
            dimension_semantics=("arbitrary",)),
    )(st2, we2, te2, po2, it2, ab2, sp_e, mv_e, hp2, bo2,
      status_table, weather_table, terrain_table, position_table,
      item_table, ability_table,
      hp_W, hp_b, boost_W, boost_b,
      wproj_bf, proj_b, ln_gamma, ln_beta)

    return out.reshape(S, B, HIDDEN).transpose(1, 0, 2)


# speedup vs baseline: 1.3128x; 1.0432x over previous
import functools

import jax
import jax.numpy as jnp
from jax import lax
from jax.experimental import pallas as pl
from jax.experimental.pallas import tpu as pltpu
from jax.experimental.pallas import tpu_sc as plsc

B, S = 1024, 50
N = B * S
HIDDEN = 1024
EPS = 1e-05

NC, NS = 2, 16
NW = NC * NS
TOK_PER_W = N // NW
CHUNK = 80
NCHUNK = TOK_PER_W // CHUNK
BIG_WIDTHS = (256, 128)

TOK_TILE = 1024
GRID = N // TOK_TILE


def _sc_gather(sp_ids_a, mv_ids_a, sp_t, mv_t):
    mesh = plsc.VectorSubcoreMesh(core_axis_name="c", subcore_axis_name="s")

    @functools.partial(
        pl.kernel,
        mesh=mesh,
        out_type=[jax.ShapeDtypeStruct((N, 128), jnp.int32),
                  jax.ShapeDtypeStruct((N, 128), jnp.float32)],
        scratch_types=[
            pltpu.VMEM((TOK_PER_W,), jnp.int32),
            pltpu.VMEM((TOK_PER_W,), jnp.int32),
            pltpu.VMEM((CHUNK, 128), jnp.int32),
            pltpu.VMEM((CHUNK, 128), jnp.int32),
            pltpu.VMEM((CHUNK, 128), jnp.int32),
            pltpu.VMEM((CHUNK, 128), jnp.float32),
            pltpu.VMEM((CHUNK, 128), jnp.float32),
            pltpu.VMEM((CHUNK, 128), jnp.float32),
            pltpu.SemaphoreType.DMA,
            pltpu.SemaphoreType.DMA,
        ],
    )
    def k(sp_ids, mv_ids, sp_hbm, mv_hbm, o_sp, o_mv,
          i_sp, i_mv, b_sp0, b_sp1, b_sp2, b_mv0, b_mv1, b_mv2, gsem, ssem):
        wid = lax.axis_index("s") * NC + lax.axis_index("c")
        base_w = pl.multiple_of(wid * TOK_PER_W, TOK_PER_W)
        pltpu.sync_copy(sp_ids.at[wid], i_sp)
        pltpu.sync_copy(mv_ids.at[wid], i_mv)
        idxs = (i_sp, i_mv)
        tables = (sp_hbm, mv_hbm)
        bufs = ((b_sp0, b_sp1, b_sp2), (b_mv0, b_mv1, b_mv2))
        outs = (o_sp, o_mv)
        DEPTH = 3

        def gather(kk, slot):
            off = pl.multiple_of(kk * CHUNK, CHUNK)
            return [pltpu.async_copy(
                tables[t].at[idxs[t].at[pl.ds(off, CHUNK)]],
                bufs[t][slot], gsem) for t in range(2)]

        def store(kk, slot):
            off = pl.multiple_of(kk * CHUNK, CHUNK)
            return [pltpu.async_copy(
                bufs[t][slot], outs[t].at[pl.ds(base_w + off, CHUNK)], ssem)
                for t in range(2)]

        gaths = [None] * NCHUNK
        stores = [[] for _ in range(DEPTH)]
        for j in range(min(DEPTH - 1, NCHUNK)):
            gaths[j] = gather(j, j % DEPTH)
        for kk in range(NCHUNK):
            slot = kk % DEPTH
            nxt = kk + DEPTH - 1
            if nxt < NCHUNK:
                for x in stores[nxt % DEPTH]:
                    x.wait()
                stores[nxt % DEPTH] = []
                gaths[nxt] = gather(nxt, nxt % DEPTH)
            for x in gaths[kk]:
                x.wait()
            stores[slot] = store(kk, slot)
        for sl in range(DEPTH):
            for x in stores[sl]:
                x.wait()

    return k(sp_ids_a, mv_ids_a, sp_t, mv_t)


def _tc_body(st_ids_ref, we_ids_ref, te_ids_ref, po_ids_ref,
             it_ids_ref, ab_ids_ref,
             sp_ref, mv_ref, hp_ref, bo_ref,
             st_t_ref, we_t_ref, te_t_ref, po_t_ref, it_t_ref, ab_t_ref,
             hp_W_ref, hp_b_ref, boost_W_ref, boost_b_ref,
             wproj_ref, proj_b_ref, gamma_ref, beta_ref, out_ref):
    f32 = jnp.float32
    bf16 = jnp.bfloat16

    def onehot_emb(ids_ref, tbl_ref, vocab):
        ids = ids_ref[0, 0, :]
        oh = (ids[:, None] == lax.broadcasted_iota(
            jnp.int32, (TOK_TILE, vocab), 1)).astype(bf16)
        return jnp.dot(oh, tbl_ref[...].astype(bf16),
                       preferred_element_type=f32)

    st_emb = onehot_emb(st_ids_ref, st_t_ref, 8)
    we_emb = onehot_emb(we_ids_ref, we_t_ref, 10)
    te_emb = onehot_emb(te_ids_ref, te_t_ref, 5)
    po_emb = onehot_emb(po_ids_ref, po_t_ref, 12)
    it_emb = onehot_emb(it_ids_ref, it_t_ref, 400)
    ab_emb = onehot_emb(ab_ids_ref, ab_t_ref, 300)
    hp_emb = hp_ref[...] * hp_W_ref[...] + hp_b_ref[...][None, :]
    bo_emb = jnp.dot(bo_ref[...], boost_W_ref[...],
                     preferred_element_type=f32) + boost_b_ref[...][None, :]

    spw = sp_ref[...]
    sp_lo = lax.bitcast_convert_type(spw << 16, f32)
    sp_hi = lax.bitcast_convert_type(spw & jnp.int32(-65536), f32)
    combined = jnp.concatenate([
        sp_lo, sp_hi,
        mv_ref[...], it_emb, ab_emb,
        hp_emb, bo_emb, st_emb, we_emb, te_emb, po_emb], axis=1).astype(bf16)

    acc = jnp.dot(combined, wproj_ref[...], preferred_element_type=f32)
    acc = acc + proj_b_ref[...][None, :]
    mean = jnp.mean(acc, axis=1, keepdims=True)
    cen = acc - mean
    var = jnp.mean(cen * cen, axis=1, keepdims=True)
    y = cen * lax.rsqrt(var + EPS)
    out_ref[...] = y * gamma_ref[...][None, :] + beta_ref[...][None, :]


def _full(shape):
    nd = len(shape)
    return pl.BlockSpec(shape, lambda i: (0,) * nd)


def kernel(species_ids, move_ids, item_ids, ability_ids, hp_values, stat_boosts,
           status_ids, weather_ids, terrain_ids, position_ids,
           species_table, move_table, item_table, ability_table,
           hp_W, hp_b, boost_W, boost_b,
           status_table, weather_table, terrain_table, position_table,
           proj_W, proj_b, ln_gamma, ln_beta):
    i32 = jnp.int32
    sp_idw = species_ids.T.reshape(NW, TOK_PER_W).astype(i32)
    mv_idw = move_ids.T.reshape(NW, TOK_PER_W).astype(i32)

    lo16 = lax.bitcast_convert_type(
        species_table[:, :128].astype(jnp.bfloat16), jnp.uint16).astype(jnp.uint32)
    hi16 = lax.bitcast_convert_type(
        species_table[:, 128:].astype(jnp.bfloat16), jnp.uint16).astype(jnp.uint32)
    sp_packed = lax.bitcast_convert_type(lo16 | (hi16 << 16), jnp.int32)
    sp_e, mv_e = _sc_gather(sp_idw, mv_idw, sp_packed, move_table)

    st2 = status_ids.T.reshape(GRID, 1, TOK_TILE).astype(i32)
    we2 = weather_ids.T.reshape(GRID, 1, TOK_TILE).astype(i32)
    te2 = terrain_ids.T.reshape(GRID, 1, TOK_TILE).astype(i32)
    po2 = position_ids.T.reshape(GRID, 1, TOK_TILE).astype(i32)
    it2 = item_ids.T.reshape(GRID, 1, TOK_TILE).astype(i32)
    ab2 = ability_ids.T.reshape(GRID, 1, TOK_TILE).astype(i32)
    hp2 = hp_values.T.reshape(N, 1)
    bo2 = stat_boosts.transpose(1, 0, 2).reshape(N, 7)
    wproj_bf = proj_W.astype(jnp.bfloat16)

    ids_spec = pl.BlockSpec((1, 1, TOK_TILE), lambda i: (i, 0, 0))

    out = pl.pallas_call(
        _tc_body,
        grid=(GRID,),
        in_specs=[
            ids_spec, ids_spec, ids_spec, ids_spec, ids_spec, ids_spec,
            pl.BlockSpec((TOK_TILE, 128), lambda i: (i, 0)),
            pl.BlockSpec((TOK_TILE, 128), lambda i: (i, 0)),
            pl.BlockSpec((TOK_TILE, 1), lambda i: (i, 0)),
            pl.BlockSpec((TOK_TILE, 7), lambda i: (i, 0)),
            _full((8, 32)), _full((10, 32)), _full((5, 32)), _full((12, 64)),
            _full((400, 64)), _full((300, 64)),
            _full((1, 32)), _full((32,)), _full((7, 32)), _full((32,)),
            _full((736, HIDDEN)), _full((HIDDEN,)),
            _full((HIDDEN,)), _full((HIDDEN,)),
        ],
        out_specs=pl.BlockSpec((TOK_TILE, HIDDEN), lambda i: (i, 0)),
        out_shape=jax.ShapeDtypeStruct((N, HIDDEN), jnp.float32),
        compiler_params=pltpu.CompilerParams(
            dimension_semantics=("arbitrary",)),
    )(st2, we2, te2, po2, it2, ab2, sp_e, mv_e, hp2, bo2,
      status_table, weather_table, terrain_table, position_table,
      item_table, ability_table,
      hp_W, hp_b, boost_W, boost_b,
      wproj_bf, proj_b, ln_gamma, ln_beta)

    return out.reshape(S, B, HIDDEN).transpose(1, 0, 2)


# speedup vs baseline: 1.3940x; 1.0619x over previous
import functools

import jax
import jax.numpy as jnp
from jax import lax
from jax.experimental import pallas as pl
from jax.experimental.pallas import tpu as pltpu
from jax.experimental.pallas import tpu_sc as plsc

B, S = 1024, 50
N = B * S
HIDDEN = 1024
EPS = 1e-05

NC, NS = 2, 16
NW = NC * NS
TOK_PER_W = N // NW
CHUNK = 80
NCHUNK = TOK_PER_W // CHUNK
BIG_WIDTHS = (256, 128)

TOK_TILE = 1024
GRID = N // TOK_TILE


def _sc_gather(sp_ids_a, mv_ids_a, sp_t, mv_t):
    mesh = plsc.VectorSubcoreMesh(core_axis_name="c", subcore_axis_name="s")

    @functools.partial(
        pl.kernel,
        mesh=mesh,
        out_type=[jax.ShapeDtypeStruct((N, 128), jnp.int32),
                  jax.ShapeDtypeStruct((N, 128), jnp.float32)],
        scratch_types=[
            pltpu.VMEM((TOK_PER_W,), jnp.int32),
            pltpu.VMEM((TOK_PER_W,), jnp.int32),
            pltpu.VMEM((CHUNK, 128), jnp.int32),
            pltpu.VMEM((CHUNK, 128), jnp.int32),
            pltpu.VMEM((CHUNK, 128), jnp.int32),
            pltpu.VMEM((CHUNK, 128), jnp.float32),
            pltpu.VMEM((CHUNK, 128), jnp.float32),
            pltpu.VMEM((CHUNK, 128), jnp.float32),
            pltpu.SemaphoreType.DMA,
            pltpu.SemaphoreType.DMA,
        ],
    )
    def k(sp_ids, mv_ids, sp_hbm, mv_hbm, o_sp, o_mv,
          i_sp, i_mv, b_sp0, b_sp1, b_sp2, b_mv0, b_mv1, b_mv2, gsem, ssem):
        wid = lax.axis_index("s") * NC + lax.axis_index("c")
        base_w = pl.multiple_of(wid * TOK_PER_W, TOK_PER_W)
        pltpu.sync_copy(sp_ids.at[wid], i_sp)
        pltpu.sync_copy(mv_ids.at[wid], i_mv)
        idxs = (i_sp, i_mv)
        tables = (sp_hbm, mv_hbm)
        bufs = ((b_sp0, b_sp1, b_sp2), (b_mv0, b_mv1, b_mv2))
        outs = (o_sp, o_mv)
        DEPTH = 3

        def gather(kk, slot):
            off = pl.multiple_of(kk * CHUNK, CHUNK)
            return [pltpu.async_copy(
                tables[t].at[idxs[t].at[pl.ds(off, CHUNK)]],
                bufs[t][slot], gsem) for t in range(2)]

        def store(kk, slot):
            off = pl.multiple_of(kk * CHUNK, CHUNK)
            return [pltpu.async_copy(
                bufs[t][slot], outs[t].at[pl.ds(base_w + off, CHUNK)], ssem)
                for t in range(2)]

        gaths = [None] * NCHUNK
        stores = [[] for _ in range(DEPTH)]
        for j in range(min(DEPTH - 1, NCHUNK)):
            gaths[j] = gather(j, j % DEPTH)
        for kk in range(NCHUNK):
            slot = kk % DEPTH
            nxt = kk + DEPTH - 1
            if nxt < NCHUNK:
                for x in stores[nxt % DEPTH]:
                    x.wait()
                stores[nxt % DEPTH] = []
                gaths[nxt] = gather(nxt, nxt % DEPTH)
            for x in gaths[kk]:
                x.wait()
            stores[slot] = store(kk, slot)
        for sl in range(DEPTH):
            for x in stores[sl]:
                x.wait()

    return k(sp_ids_a, mv_ids_a, sp_t, mv_t)


def _tc_body(st_ids_ref, we_ids_ref, te_ids_ref, po_ids_ref,
             it_ids_ref, ab_ids_ref,
             sp_ref, mv_ref, hbo_ref,
             st_t_ref, we_t_ref, te_t_ref, po_t_ref, it_t_ref, ab_t_ref,
             hb_W_ref, hb_b_ref,
             wproj_ref, proj_b_ref, gamma_ref, beta_ref, out_ref):
    f32 = jnp.float32
    bf16 = jnp.bfloat16

    def onehot_emb(ids_ref, tbl_ref, vocab):
        ids = ids_ref[0, 0, :]
        oh = (ids[:, None] == lax.broadcasted_iota(
            jnp.int32, (TOK_TILE, vocab), 1)).astype(bf16)
        return jnp.dot(oh, tbl_ref[...].astype(bf16),
                       preferred_element_type=f32)

    st_emb = onehot_emb(st_ids_ref, st_t_ref, 8)
    we_emb = onehot_emb(we_ids_ref, we_t_ref, 10)
    te_emb = onehot_emb(te_ids_ref, te_t_ref, 5)
    po_emb = onehot_emb(po_ids_ref, po_t_ref, 12)
    it_emb = onehot_emb(it_ids_ref, it_t_ref, 400)
    ab_emb = onehot_emb(ab_ids_ref, ab_t_ref, 300)
    hb_emb = lax.dot_general(
        hbo_ref[...], hb_W_ref[...],
        dimension_numbers=(((0,), (0,)), ((), ())),
        preferred_element_type=f32) + hb_b_ref[...][None, :]

    spw = sp_ref[...]
    sp_lo = lax.bitcast_convert_type(spw << 16, f32)
    sp_hi = lax.bitcast_convert_type(spw & jnp.int32(-65536), f32)
    combined = jnp.concatenate([
        sp_lo, sp_hi,
        mv_ref[...], it_emb, ab_emb,
        hb_emb, st_emb, we_emb, te_emb, po_emb], axis=1).astype(bf16)

    acc = jnp.dot(combined, wproj_ref[...], preferred_element_type=f32)
    acc = acc + proj_b_ref[...][None, :]
    mean = jnp.mean(acc, axis=1, keepdims=True)
    cen = acc - mean
    var = jnp.mean(cen * cen, axis=1, keepdims=True)
    y = cen * lax.rsqrt(var + EPS)
    out_ref[...] = y * gamma_ref[...][None, :] + beta_ref[...][None, :]


def _full(shape):
    nd = len(shape)
    return pl.BlockSpec(shape, lambda i: (0,) * nd)


def kernel(species_ids, move_ids, item_ids, ability_ids, hp_values, stat_boosts,
           status_ids, weather_ids, terrain_ids, position_ids,
           species_table, move_table, item_table, ability_table,
           hp_W, hp_b, boost_W, boost_b,
           status_table, weather_table, terrain_table, position_table,
           proj_W, proj_b, ln_gamma, ln_beta):
    i32 = jnp.int32
    sp_idw = species_ids.T.reshape(NW, TOK_PER_W).astype(i32)
    mv_idw = move_ids.T.reshape(NW, TOK_PER_W).astype(i32)

    lo16 = lax.bitcast_convert_type(
        species_table[:, :128].astype(jnp.bfloat16), jnp.uint16).astype(jnp.uint32)
    hi16 = lax.bitcast_convert_type(
        species_table[:, 128:].astype(jnp.bfloat16), jnp.uint16).astype(jnp.uint32)
    sp_packed = lax.bitcast_convert_type(lo16 | (hi16 << 16), jnp.int32)
    sp_e, mv_e = _sc_gather(sp_idw, mv_idw, sp_packed, move_table)

    st2 = status_ids.T.reshape(GRID, 1, TOK_TILE).astype(i32)
    we2 = weather_ids.T.reshape(GRID, 1, TOK_TILE).astype(i32)
    te2 = terrain_ids.T.reshape(GRID, 1, TOK_TILE).astype(i32)
    po2 = position_ids.T.reshape(GRID, 1, TOK_TILE).astype(i32)
    it2 = item_ids.T.reshape(GRID, 1, TOK_TILE).astype(i32)
    ab2 = ability_ids.T.reshape(GRID, 1, TOK_TILE).astype(i32)
    hbo = jnp.concatenate([hp_values.T.reshape(1, N),
                           stat_boosts.transpose(2, 1, 0).reshape(7, N)],
                          axis=0)
    hb_W = jnp.zeros((8, 64), jnp.float32)
    hb_W = hb_W.at[0, :32].set(hp_W[0]).at[1:, 32:].set(boost_W)
    hb_b = jnp.concatenate([hp_b, boost_b])
    wproj_bf = proj_W.astype(jnp.bfloat16)

    ids_spec = pl.BlockSpec((1, 1, TOK_TILE), lambda i: (i, 0, 0))

    out = pl.pallas_call(
        _tc_body,
        grid=(GRID,),
        in_specs=[
            ids_spec, ids_spec, ids_spec, ids_spec, ids_spec, ids_spec,
            pl.BlockSpec((TOK_TILE, 128), lambda i: (i, 0)),
            pl.BlockSpec((TOK_TILE, 128), lambda i: (i, 0)),
            pl.BlockSpec((8, TOK_TILE), lambda i: (0, i)),
            _full((8, 32)), _full((10, 32)), _full((5, 32)), _full((12, 64)),
            _full((400, 64)), _full((300, 64)),
            _full((8, 64)), _full((64,)),
            _full((736, HIDDEN)), _full((HIDDEN,)),
            _full((HIDDEN,)), _full((HIDDEN,)),
        ],
        out_specs=pl.BlockSpec((TOK_TILE, HIDDEN), lambda i: (i, 0)),
        out_shape=jax.ShapeDtypeStruct((N, HIDDEN), jnp.float32),
        compiler_params=pltpu.CompilerParams(
            dimension_semantics=("arbitrary",)),
    )(st2, we2, te2, po2, it2, ab2, sp_e, mv_e, hbo,
      status_table, weather_table, terrain_table, position_table,
      item_table, ability_table,
      hb_W, hb_b,
      wproj_bf, proj_b, ln_gamma, ln_beta)

    return out.reshape(S, B, HIDDEN).transpose(1, 0, 2)


# speedup vs baseline: 1.4035x; 1.0068x over previous
import functools

import jax
import jax.numpy as jnp
from jax import lax
from jax.experimental import pallas as pl
from jax.experimental.pallas import tpu as pltpu
from jax.experimental.pallas import tpu_sc as plsc

B, S = 1024, 50
N = B * S
HIDDEN = 1024
EPS = 1e-05

NC, NS = 2, 16
NW = NC * NS
TOK_PER_W = N // NW
CHUNK = 80
NCHUNK = TOK_PER_W // CHUNK
BIG_WIDTHS = (256, 128)

TOK_TILE = 1024
GRID = N // TOK_TILE


def _sc_gather(sp_ids_a, mv_ids_a, sp_t, mv_t):
    mesh = plsc.VectorSubcoreMesh(core_axis_name="c", subcore_axis_name="s")

    @functools.partial(
        pl.kernel,
        mesh=mesh,
        out_type=[jax.ShapeDtypeStruct((N, 128), jnp.int32),
                  jax.ShapeDtypeStruct((N, 128), jnp.float32)],
        scratch_types=[
            pltpu.VMEM((TOK_PER_W,), jnp.int32),
            pltpu.VMEM((TOK_PER_W,), jnp.int32),
            pltpu.VMEM((CHUNK, 128), jnp.int32),
            pltpu.VMEM((CHUNK, 128), jnp.int32),
            pltpu.VMEM((CHUNK, 128), jnp.int32),
            pltpu.VMEM((CHUNK, 128), jnp.float32),
            pltpu.VMEM((CHUNK, 128), jnp.float32),
            pltpu.VMEM((CHUNK, 128), jnp.float32),
            pltpu.SemaphoreType.DMA,
            pltpu.SemaphoreType.DMA,
        ],
    )
    def k(sp_ids, mv_ids, sp_hbm, mv_hbm, o_sp, o_mv,
          i_sp, i_mv, b_sp0, b_sp1, b_sp2, b_mv0, b_mv1, b_mv2, gsem, ssem):
        wid = lax.axis_index("s") * NC + lax.axis_index("c")
        base_w = pl.multiple_of(wid * TOK_PER_W, TOK_PER_W)
        pltpu.sync_copy(sp_ids.at[wid], i_sp)
        pltpu.sync_copy(mv_ids.at[wid], i_mv)
        idxs = (i_sp, i_mv)
        tables = (sp_hbm, mv_hbm)
        bufs = ((b_sp0, b_sp1, b_sp2), (b_mv0, b_mv1, b_mv2))
        outs = (o_sp, o_mv)
        DEPTH = 3

        def gather(kk, slot):
            off = pl.multiple_of(kk * CHUNK, CHUNK)
            return [pltpu.async_copy(
                tables[t].at[idxs[t].at[pl.ds(off, CHUNK)]],
                bufs[t][slot], gsem) for t in range(2)]

        def store(kk, slot):
            off = pl.multiple_of(kk * CHUNK, CHUNK)
            return [pltpu.async_copy(
                bufs[t][slot], outs[t].at[pl.ds(base_w + off, CHUNK)], ssem)
                for t in range(2)]

        gaths = [None] * NCHUNK
        stores = [[] for _ in range(DEPTH)]
        for j in range(min(DEPTH - 1, NCHUNK)):
            gaths[j] = gather(j, j % DEPTH)
        for kk in range(NCHUNK):
            slot = kk % DEPTH
            nxt = kk + DEPTH - 1
            if nxt < NCHUNK:
                for x in stores[nxt % DEPTH]:
                    x.wait()
                stores[nxt % DEPTH] = []
                gaths[nxt] = gather(nxt, nxt % DEPTH)
            for x in gaths[kk]:
                x.wait()
            stores[slot] = store(kk, slot)
        for sl in range(DEPTH):
            for x in stores[sl]:
                x.wait()

    return k(sp_ids_a, mv_ids_a, sp_t, mv_t)


def _tc_body(ids6_ref,
             sp_ref, mv_ref, hbo_ref,
             st_t_ref, we_t_ref, te_t_ref, po_t_ref, it_t_ref, ab_t_ref,
             hb_W_ref, hb_b_ref,
             wproj_ref, proj_b_ref, gamma_ref, beta_ref, out_ref):
    f32 = jnp.float32
    bf16 = jnp.bfloat16

    def onehot_emb(row, tbl_ref, vocab):
        ids = ids6_ref[0, row, :]
        oh = (ids[:, None] == lax.broadcasted_iota(
            jnp.int32, (TOK_TILE, vocab), 1)).astype(bf16)
        return jnp.dot(oh, tbl_ref[...].astype(bf16),
                       preferred_element_type=f32)

    st_emb = onehot_emb(0, st_t_ref, 8)
    we_emb = onehot_emb(1, we_t_ref, 10)
    te_emb = onehot_emb(2, te_t_ref, 5)
    po_emb = onehot_emb(3, po_t_ref, 12)
    it_emb = onehot_emb(4, it_t_ref, 400)
    ab_emb = onehot_emb(5, ab_t_ref, 300)
    hb_emb = lax.dot_general(
        hbo_ref[...], hb_W_ref[...],
        dimension_numbers=(((0,), (0,)), ((), ())),
        preferred_element_type=f32) + hb_b_ref[...][None, :]

    spw = sp_ref[...]
    sp_lo = lax.bitcast_convert_type(spw << 16, f32)
    sp_hi = lax.bitcast_convert_type(spw & jnp.int32(-65536), f32)
    combined = jnp.concatenate([
        sp_lo, sp_hi,
        mv_ref[...], it_emb, ab_emb,
        hb_emb, st_emb, we_emb, te_emb, po_emb], axis=1).astype(bf16)

    acc = jnp.dot(combined, wproj_ref[...], preferred_element_type=f32)
    acc = acc + proj_b_ref[...][None, :]
    mean = jnp.mean(acc, axis=1, keepdims=True)
    cen = acc - mean
    var = jnp.mean(cen * cen, axis=1, keepdims=True)
    y = cen * lax.rsqrt(var + EPS)
    out_ref[...] = y * gamma_ref[...][None, :] + beta_ref[...][None, :]


def _full(shape):
    nd = len(shape)
    return pl.BlockSpec(shape, lambda i: (0,) * nd)


def kernel(species_ids, move_ids, item_ids, ability_ids, hp_values, stat_boosts,
           status_ids, weather_ids, terrain_ids, position_ids,
           species_table, move_table, item_table, ability_table,
           hp_W, hp_b, boost_W, boost_b,
           status_table, weather_table, terrain_table, position_table,
           proj_W, proj_b, ln_gamma, ln_beta):
    i32 = jnp.int32
    sp_idw = species_ids.T.reshape(NW, TOK_PER_W).astype(i32)
    mv_idw = move_ids.T.reshape(NW, TOK_PER_W).astype(i32)

    lo16 = lax.bitcast_convert_type(
        species_table[:, :128].astype(jnp.bfloat16), jnp.uint16).astype(jnp.uint32)
    hi16 = lax.bitcast_convert_type(
        species_table[:, 128:].astype(jnp.bfloat16), jnp.uint16).astype(jnp.uint32)
    sp_packed = lax.bitcast_convert_type(lo16 | (hi16 << 16), jnp.int32)
    sp_e, mv_e = _sc_gather(sp_idw, mv_idw, sp_packed, move_table)

    ids6 = jnp.stack([status_ids, weather_ids, terrain_ids, position_ids,
                      item_ids, ability_ids]).astype(i32).transpose(2, 0, 1)
    hbo = jnp.concatenate([hp_values.T.reshape(1, N),
                           stat_boosts.transpose(2, 1, 0).reshape(7, N)],
                          axis=0)
    hb_W = jnp.zeros((8, 64), jnp.float32)
    hb_W = hb_W.at[0, :32].set(hp_W[0]).at[1:, 32:].set(boost_W)
    hb_b = jnp.concatenate([hp_b, boost_b])
    wproj_bf = proj_W.astype(jnp.bfloat16)

    ids_spec = pl.BlockSpec((1, 6, TOK_TILE), lambda i: (i, 0, 0))

    out = pl.pallas_call(
        _tc_body,
        grid=(GRID,),
        in_specs=[
            ids_spec,
            pl.BlockSpec((TOK_TILE, 128), lambda i: (i, 0)),
            pl.BlockSpec((TOK_TILE, 128), lambda i: (i, 0)),
            pl.BlockSpec((8, TOK_TILE), lambda i: (0, i)),
            _full((8, 32)), _full((10, 32)), _full((5, 32)), _full((12, 64)),
            _full((400, 64)), _full((300, 64)),
            _full((8, 64)), _full((64,)),
            _full((736, HIDDEN)), _full((HIDDEN,)),
            _full((HIDDEN,)), _full((HIDDEN,)),
        ],
        out_specs=pl.BlockSpec((TOK_TILE, HIDDEN), lambda i: (i, 0)),
        out_shape=jax.ShapeDtypeStruct((N, HIDDEN), jnp.float32),
        compiler_params=pltpu.CompilerParams(
            dimension_semantics=("arbitrary",)),
    )(ids6, sp_e, mv_e, hbo,
      status_table, weather_table, terrain_table, position_table,
      item_table, ability_table,
      hb_W, hb_b,
      wproj_bf, proj_b, ln_gamma, ln_beta)

    return out.reshape(S, B, HIDDEN).transpose(1, 0, 2)
